# Initial kernel scaffold; baseline (speedup 1.0000x reference)
import jax
import jax.numpy as jnp
from jax.experimental import pallas as pl


def kernel(x, embedding):
    raise NotImplementedError("write your pallas kernel here")


# speedup vs baseline: 1.1014x; 1.1014x over previous
import functools

import jax
import jax.numpy as jnp
from jax import lax
from jax.experimental import pallas as pl
from jax.experimental.pallas import tpu as pltpu
from jax.experimental.pallas import tpu_sc as plsc

B = 16384 * 50
D = 32
NC, NS = 2, 16
NW = NC * NS
BPW = B // NW
C = 1600
NCHUNK = BPW // C

_mesh = plsc.VectorSubcoreMesh(core_axis_name="c", subcore_axis_name="s")


@functools.partial(
    pl.kernel,
    out_type=jax.ShapeDtypeStruct((B, D), jnp.float32),
    mesh=_mesh,
    scratch_types=[
        pltpu.VMEM((C,), jnp.int32),
        pltpu.VMEM((C, D), jnp.float32),
        pltpu.SemaphoreType.DMA,
    ],
    compiler_params=pltpu.CompilerParams(use_tc_tiling_on_sc=False),
)
def _gather_kernel(idx_hbm, table_hbm, out_hbm, idx_v, rows_v, sem):
    wid = lax.axis_index("s") * NC + lax.axis_index("c")
    base = wid * BPW
    for c in range(NCHUNK):
        off = base + c * C
        pltpu.sync_copy(idx_hbm.at[pl.ds(off, C)], idx_v)
        pltpu.async_copy(table_hbm.at[idx_v], rows_v, sem).wait()
        pltpu.sync_copy(rows_v, out_hbm.at[pl.ds(off, C)])


def kernel(x, embedding):
    flat = x.reshape(-1)
    out = _gather_kernel(flat, embedding)
    return out.reshape(x.shape + (D,))


# speedup vs baseline: 1.1126x; 1.0102x over previous
import functools

import jax
import jax.numpy as jnp
from jax import lax
from jax.experimental import pallas as pl
from jax.experimental.pallas import tpu as pltpu
from jax.experimental.pallas import tpu_sc as plsc

B = 16384 * 50
D = 32
NC, NS = 2, 16
NW = NC * NS
BPW = B // NW
C = 1600
NCHUNK = BPW // C

_mesh = plsc.VectorSubcoreMesh(core_axis_name="c", subcore_axis_name="s")


@functools.partial(
    pl.kernel,
    out_type=jax.ShapeDtypeStruct((B, D), jnp.float32),
    mesh=_mesh,
    scratch_types=[
        pltpu.VMEM((4, C), jnp.int32),
        pltpu.VMEM((2, C, D), jnp.float32),
        pltpu.SemaphoreType.DMA,
        pltpu.SemaphoreType.DMA,
        pltpu.SemaphoreType.DMA,
        pltpu.SemaphoreType.DMA,
        pltpu.SemaphoreType.DMA,
        pltpu.SemaphoreType.DMA,
        pltpu.SemaphoreType.DMA,
        pltpu.SemaphoreType.DMA,
    ],
    compiler_params=pltpu.CompilerParams(use_tc_tiling_on_sc=False),
)
def _gather_kernel(idx_hbm, table_hbm, out_hbm, idx_v, rows_v,
                   si0, si1, si2, si3, sg0, sg1, so0, so1):
    si = [si0, si1, si2, si3]
    sg = [sg0, sg1]
    so = [so0, so1]
    wid = lax.axis_index("s") * NC + lax.axis_index("c")
    base = wid * BPW

    idx_d = [None] * NCHUNK
    gat_d = [None] * NCHUNK
    out_d = [None] * NCHUNK

    for c in range(min(4, NCHUNK)):
        idx_d[c] = pltpu.async_copy(
            idx_hbm.at[pl.ds(base + c * C, C)], idx_v.at[c & 3], si[c & 3])

    for c in range(NCHUNK):
        rb, ib = c & 1, c & 3
        idx_d[c].wait()
        if c >= 2:
            out_d[c - 2].wait()
        gat_d[c] = pltpu.async_copy(
            table_hbm.at[idx_v.at[ib]], rows_v.at[rb], sg[rb])
        if c >= 1:
            gat_d[c - 1].wait()
            out_d[c - 1] = pltpu.async_copy(
                rows_v.at[1 - rb],
                out_hbm.at[pl.ds(base + (c - 1) * C, C)], so[1 - rb])
            if c + 3 < NCHUNK:
                idx_d[c + 3] = pltpu.async_copy(
                    idx_hbm.at[pl.ds(base + (c + 3) * C, C)],
                    idx_v.at[(c + 3) & 3], si[(c + 3) & 3])

    last = NCHUNK - 1
    gat_d[last].wait()
    out_d[last] = pltpu.async_copy(
        rows_v.at[last & 1], out_hbm.at[pl.ds(base + last * C, C)],
        so[last & 1])
    out_d[last - 1].wait()
    out_d[last].wait()


def kernel(x, embedding):
    flat = x.reshape(-1)
    out = _gather_kernel(flat, embedding)
    return out.reshape(x.shape + (D,))


# speedup vs baseline: 1.7767x; 1.5969x over previous
import functools

import jax
import jax.numpy as jnp
from jax import lax
from jax.experimental import pallas as pl
from jax.experimental.pallas import tpu as pltpu
from jax.experimental.pallas import tpu_sc as plsc

NROW = 16384
NCOL = 50
D = 32
NC, NS = 2, 16
NW = NC * NS
RPW = NROW // NW
R = 64
NCHUNK = RPW // R
C = R * NCOL

_mesh = plsc.VectorSubcoreMesh(core_axis_name="c", subcore_axis_name="s")


@functools.partial(
    pl.kernel,
    out_type=jax.ShapeDtypeStruct((NROW, NCOL, D), jnp.float32),
    mesh=_mesh,
    scratch_types=[
        pltpu.VMEM((R, NCOL), jnp.int32),
        pltpu.VMEM((C,), jnp.int32),
        pltpu.VMEM((C, D), jnp.float32),
        pltpu.SemaphoreType.DMA,
        pltpu.SemaphoreType.DMA,
    ],
    compiler_params=pltpu.CompilerParams(
        use_tc_tiling_on_sc=False, needs_layout_passes=False),
)
def _gather_kernel(x_hbm, table_hbm, out_hbm, xchunk_v, idx_v, rows_v, sem,
                   sem_wb):
    wid = lax.axis_index("s") * NC + lax.axis_index("c")
    base_row = wid * RPW
    lanes = lax.iota(jnp.int32, 16)

    for c in range(NCHUNK):
        r0 = base_row + c * R
        pltpu.sync_copy(x_hbm.at[pl.ds(r0, R), :], xchunk_v)

        def transpose_step(t, _):
            k = t * 16 + lanes
            r = jnp.bitwise_and(k, R - 1)
            j = jnp.right_shift(k, 6)
            vals = plsc.load_gather(xchunk_v, [r, j])
            idx_v[pl.ds(t * 16, 16)] = vals
            return _

        lax.fori_loop(0, C // 16, transpose_step, 0)

        pltpu.async_copy(table_hbm.at[idx_v], rows_v, sem).wait()

        wb = [
            pltpu.async_copy(
                rows_v.at[pl.ds(j * R, R), :],
                out_hbm.at[pl.ds(r0, R), j, :], sem_wb)
            for j in range(NCOL)
        ]
        for d in wb:
            d.wait()


def kernel(x, embedding):
    return _gather_kernel(x, embedding)
